# baseline (speedup 1.0000x reference)
import functools

import jax
import jax.numpy as jnp
from jax import lax
from jax.experimental import pallas as pl
from jax.experimental.pallas import tpu as pltpu
from jax.experimental.pallas import tpu_sc as plsc

NUM_EMB = 1000000
D = 64
B = 16384
H = 20
L = 16
QB = 4
QI = QB * H
NV = QI // L

NC = 2
NS = 16
NW = NC * NS
BPW = B // NW
PER_W = BPW * H
NSTEP = BPW // QB


def _emb_kernel(idx_hbm, table_hbm, out_hbm, idx_v, sem_i, sem):
    wid = lax.axis_index("s") * NC + lax.axis_index("c")
    b0 = wid * BPW
    pltpu.async_copy(idx_hbm.at[wid], idx_v, sem_i).wait()

    def step(s, carry):
        b_base = b0 + s * QB
        flat = s * QI
        for q in range(NV):
            v = idx_v[pl.ds(flat + q * L, L)]
            for l in range(L):
                j = q * L + l
                r = v[l]
                pltpu.async_copy(
                    table_hbm.at[r], out_hbm.at[b_base + j // H, j % H], sem)
        return carry

    lax.fori_loop(0, NSTEP, step, 0)

    def drain(i, carry):
        pltpu.make_async_copy(table_hbm.at[0], out_hbm.at[b0, 0], sem).wait()
        return carry

    lax.fori_loop(0, PER_W, drain, 0)


@jax.jit
def kernel(input_ids, table):
    idx = jnp.reshape(input_ids.astype(jnp.int32), (NW, PER_W))
    mesh = plsc.VectorSubcoreMesh(core_axis_name="c", subcore_axis_name="s")
    run = functools.partial(
        pl.kernel,
        mesh=mesh,
        out_type=jax.ShapeDtypeStruct((B, H, D), jnp.float32),
        scratch_types=[
            pltpu.VMEM((PER_W,), jnp.int32),
            pltpu.SemaphoreType.DMA,
            pltpu.SemaphoreType.DMA,
        ],
    )(_emb_kernel)
    return run(idx, table)


# speedup vs baseline: 6.3158x; 6.3158x over previous
import functools

import jax
import jax.numpy as jnp
from jax import lax
from jax.experimental import pallas as pl
from jax.experimental.pallas import tpu as pltpu
from jax.experimental.pallas import tpu_sc as plsc

NUM_EMB = 1000000
D = 64
B = 16384
H = 20
TOTAL = B * H

NC = 2
NS = 16
NW = NC * NS
PER_W = TOTAL // NW
CHUNK = 128
NCH = PER_W // CHUNK
NBUF = 4
GROUPS = NCH // NBUF

PACK_ROWS = 5000
HALF = NUM_EMB // 2


def _pack_kernel(lo_ref, hi_ref, o_ref):
    o_ref[...] = jnp.concatenate([lo_ref[...], hi_ref[...]], axis=1)


def _emb_kernel(idx_hbm, table_hbm, out_hbm, idx_v, *scr):
    rows = scr[:NBUF]
    sem_idx = scr[NBUF]
    gsem = scr[NBUF + 1:NBUF + 1 + NBUF]
    wsem = scr[NBUF + 1 + NBUF:]
    wid = lax.axis_index("s") * NC + lax.axis_index("c")
    base = wid * PER_W
    pltpu.async_copy(idx_hbm.at[wid], idx_v, sem_idx).wait()

    def gather(c, b):
        pltpu.async_copy(table_hbm.at[idx_v.at[c]], rows[b], gsem[b])

    def wb_start(c, b):
        pltpu.async_copy(rows[b], out_hbm.at[pl.ds(base + c * CHUNK, CHUNK)],
                         wsem[b])

    def drain(sem, buf):
        pltpu.make_async_copy(table_hbm.at[pl.ds(0, CHUNK)], buf, sem).wait()

    for b in range(NBUF):
        gather(b, b)

    def body(step, carry):
        for b in range(NBUF):
            c = step * NBUF + b
            drain(gsem[b], rows[b])
            wb_start(c, b)
            drain(wsem[b], rows[b])
            gather(c + NBUF, b)
        return carry

    lax.fori_loop(0, GROUPS - 1, body, 0)

    for b in range(NBUF):
        c = (GROUPS - 1) * NBUF + b
        drain(gsem[b], rows[b])
        wb_start(c, b)
        drain(wsem[b], rows[b])


@jax.jit
def kernel(input_ids, table):
    packed = pl.pallas_call(
        _pack_kernel,
        grid=(HALF // PACK_ROWS,),
        in_specs=[
            pl.BlockSpec((PACK_ROWS, D), lambda i: (i, 0)),
            pl.BlockSpec((PACK_ROWS, D), lambda i: (i + HALF // PACK_ROWS, 0)),
        ],
        out_specs=pl.BlockSpec((PACK_ROWS, 2 * D), lambda i: (i, 0)),
        out_shape=jax.ShapeDtypeStruct((HALF, 2 * D), jnp.float32),
    )(table, table)
    flat_table = jnp.reshape(packed, (NUM_EMB, D))

    ids = input_ids.astype(jnp.int32)
    q = jnp.where(ids < HALF, 2 * ids, 2 * ids - (NUM_EMB - 1))
    idx = jnp.reshape(q, (NW, NCH, CHUNK))
    mesh = plsc.VectorSubcoreMesh(core_axis_name="c", subcore_axis_name="s")
    run = functools.partial(
        pl.kernel,
        mesh=mesh,
        out_type=jax.ShapeDtypeStruct((TOTAL, D), jnp.float32),
        scratch_types=(
            [pltpu.VMEM((NCH, CHUNK), jnp.int32)]
            + [pltpu.VMEM((CHUNK, D), jnp.float32) for _ in range(NBUF)]
            + [pltpu.SemaphoreType.DMA] * (1 + 2 * NBUF)
        ),
        compiler_params=pltpu.CompilerParams(use_tc_tiling_on_sc=False),
    )(_emb_kernel)
    out = run(idx, flat_table)
    return jnp.reshape(out, (B, H, D))


# speedup vs baseline: 7.0542x; 1.1169x over previous
import functools

import jax
import jax.numpy as jnp
from jax import lax
from jax.experimental import pallas as pl
from jax.experimental.pallas import tpu as pltpu
from jax.experimental.pallas import tpu_sc as plsc

NUM_EMB = 1000000
D = 64
B = 16384
H = 20
TOTAL = B * H

NC = 2
NS = 16
NW = NC * NS
PER_W = TOTAL // NW
CHUNK = 128
NCH = PER_W // CHUNK
NBUF = 4
GROUPS = NCH // NBUF

PACK_ROWS = 5000
HALF = NUM_EMB // 2


def _pack_kernel(lo_ref, hi_ref, o_ref):
    o_ref[...] = jnp.concatenate([lo_ref[...], hi_ref[...]], axis=1)


def _emb_kernel(idx_hbm, table_hbm, out_hbm, idx_v, *scr):
    rows = scr[:NBUF]
    sem_idx = scr[NBUF]
    gsem = scr[NBUF + 1:NBUF + 1 + NBUF]
    wsem = scr[NBUF + 1 + NBUF:]
    wid = lax.axis_index("s") * NC + lax.axis_index("c")
    base = wid * PER_W
    pltpu.async_copy(idx_hbm.at[wid], idx_v, sem_idx).wait()

    def gather(c, b):
        pltpu.async_copy(table_hbm.at[idx_v.at[c]], rows[b], gsem[b])

    def wb_start(c, b):
        pltpu.async_copy(rows[b], out_hbm.at[pl.ds(base + c * CHUNK, CHUNK)],
                         wsem[b])

    def drain(sem, buf):
        pltpu.make_async_copy(table_hbm.at[pl.ds(0, CHUNK)], buf, sem).wait()

    for b in range(NBUF):
        gather(b, b)

    def body(step, carry):
        for b in range(NBUF):
            c = step * NBUF + b
            drain(gsem[b], rows[b])
            wb_start(c, b)
            drain(wsem[b], rows[b])
            gather(c + NBUF, b)
        return carry

    lax.fori_loop(0, GROUPS - 1, body, 0)

    for b in range(NBUF):
        c = (GROUPS - 1) * NBUF + b
        drain(gsem[b], rows[b])
        wb_start(c, b)
        drain(wsem[b], rows[b])


@jax.jit
def kernel(input_ids, table):
    packed = jnp.concatenate([table[:HALF], table[HALF:]], axis=1)
    flat_table = jnp.reshape(packed, (NUM_EMB, D))

    ids = input_ids.astype(jnp.int32)
    q = jnp.where(ids < HALF, 2 * ids, 2 * ids - (NUM_EMB - 1))
    idx = jnp.reshape(q, (NW, NCH, CHUNK))
    mesh = plsc.VectorSubcoreMesh(core_axis_name="c", subcore_axis_name="s")
    run = functools.partial(
        pl.kernel,
        mesh=mesh,
        out_type=jax.ShapeDtypeStruct((TOTAL, D), jnp.float32),
        scratch_types=(
            [pltpu.VMEM((NCH, CHUNK), jnp.int32)]
            + [pltpu.VMEM((CHUNK, D), jnp.float32) for _ in range(NBUF)]
            + [pltpu.SemaphoreType.DMA] * (1 + 2 * NBUF)
        ),
        compiler_params=pltpu.CompilerParams(use_tc_tiling_on_sc=False),
    )(_emb_kernel)
    out = run(idx, flat_table)
    return jnp.reshape(out, (B, H, D))
